# baseline (speedup 1.0000x reference)
import functools

import jax
import jax.numpy as jnp
from jax import lax
from jax.experimental import pallas as pl
from jax.experimental.pallas import tpu as pltpu
from jax.experimental.pallas import tpu_sc as plsc

_N = 10000
_E = 160000
_DIN = 256
_H = 512
_C = 4
_CW = _H // _C
_NS = 16
_BATCH = 80
_ZB = 40
_NPAD = 10240
_RPT = _NPAD // _NS
_EPT = _E // _NS
_BN = 1000
_NB = _N // _BN


_sc_mesh = plsc.VectorSubcoreMesh(core_axis_name="c", subcore_axis_name="s")


_NBAT = _EPT // _BATCH


@functools.partial(
    pl.kernel,
    mesh=_sc_mesh,
    out_type=jax.ShapeDtypeStruct((_C * _NPAD, _CW), jnp.float32),
    scratch_types=[
        pltpu.VMEM((_EPT,), jnp.int32),
        pltpu.VMEM((_BATCH,), jnp.int32),
        pltpu.VMEM((_BATCH,), jnp.int32),
        pltpu.VMEM((_BATCH,), jnp.int32),
        pltpu.VMEM((_BATCH, _CW), jnp.float32),
        pltpu.VMEM((_BATCH, _CW), jnp.float32),
        pltpu.VMEM((_BATCH, _CW), jnp.float32),
        pltpu.VMEM_SHARED((_NPAD, _CW), jnp.float32),
        pltpu.SemaphoreType.DMA,
        pltpu.SemaphoreType.DMA,
        pltpu.SemaphoreType.DMA,
        pltpu.SemaphoreType.DMA,
        pltpu.SemaphoreType.DMA,
        pltpu.SemaphoreType.DMA,
        pltpu.SemaphoreType.DMA,
        pltpu.SemaphoreType.DMA,
        pltpu.SemaphoreType.DMA,
        pltpu.SemaphoreType.DMA,
        pltpu.SemaphoreType.DMA,
    ],
)
def _sc_scatter(h4, src, dst, out, src_off, dst_v0, dst_v1, dst_v2, rows0,
                rows1, rows2, acc_sh, sem0, sem1, sem2, semd0,
                semd1, semd2, sems0, sems1, sems2, semw0, semw1):
    cid = lax.axis_index("c")
    sid = lax.axis_index("s")
    ebase = sid * _EPT
    rbase = sid * _RPT
    z16 = jnp.zeros((16,), jnp.float32)
    slots = ((dst_v0, rows0, sem0, semd0, sems0),
             (dst_v1, rows1, sem1, semd1, sems1),
             (dst_v2, rows2, sem2, semd2, sems2))
    wrows = (rows0, rows1)
    wsems = (semw0, semw1)

    def dcopy(b, dv, sem):
        return pltpu.make_async_copy(
            dst.at[pl.ds(ebase + b * _BATCH, _BATCH)], dv, sem)

    def gcopy(b, rows, sem):
        return pltpu.make_async_copy(
            h4.at[src_off.at[pl.ds(b * _BATCH, _BATCH)]], rows, sem)

    def issue(b, slot):
        dcopy(b, slot[0], slot[3]).start()
        gcopy(b, slot[1], slot[2]).start()

    def drain_scatter(b, slot):
        gcopy(b, slot[1], slot[2]).wait()
        dcopy(b, slot[0], slot[3]).wait()
        pltpu.async_copy(slot[1], acc_sh.at[slot[0]], slot[4], add=True)

    def swait(slot):
        pltpu.make_async_copy(slot[1], acc_sh.at[slot[0]], slot[4]).wait()

    def prep(b, slot):
        swait(slot)
        issue(b, slot)

    for ci in range(2):
        chunk = cid * 2 + ci
        row_off = chunk * _N

        def zrow(i, carry):
            for j in range(_CW // 16):
                rows2[i, pl.ds(j * 16, 16)] = z16
            return carry

        lax.fori_loop(0, _BATCH, zrow, 0)

        def zcopy(k, carry):
            pltpu.sync_copy(rows2,
                            acc_sh.at[pl.ds(rbase + k * _BATCH, _BATCH)])
            return carry

        lax.fori_loop(0, _RPT // _BATCH, zcopy, 0)

        pltpu.sync_copy(src.at[pl.ds(ebase, _EPT)], src_off)

        def abody(k, carry):
            sl = pl.ds(k * 16, 16)
            src_off[sl] = src_off[sl] + row_off
            return carry

        lax.fori_loop(0, _EPT // 16, abody, 0)
        plsc.subcore_barrier()

        issue(0, slots[0])
        issue(1, slots[1])
        issue(2, slots[2])

        def pbody(p, carry):
            b0 = 3 * p
            drain_scatter(b0, slots[0])
            prep(b0 + 3, slots[0])
            drain_scatter(b0 + 1, slots[1])
            prep(b0 + 4, slots[1])
            drain_scatter(b0 + 2, slots[2])
            prep(b0 + 5, slots[2])
            return carry

        lax.fori_loop(0, (_NBAT - 5) // 3, pbody, 0)
        drain_scatter(_NBAT - 5, slots[0])
        prep(_NBAT - 2, slots[0])
        drain_scatter(_NBAT - 4, slots[1])
        prep(_NBAT - 1, slots[1])
        drain_scatter(_NBAT - 3, slots[2])
        drain_scatter(_NBAT - 2, slots[0])
        drain_scatter(_NBAT - 1, slots[1])
        for sl in slots:
            swait(sl)
        plsc.subcore_barrier()

        woff = chunk * _NPAD + rbase
        nwb = _RPT // _BATCH

        def wdesc(k, h):
            return pltpu.make_async_copy(
                wrows[h], out.at[pl.ds(woff + k * _BATCH, _BATCH)], wsems[h])

        for k in range(nwb):
            h = k % 2
            if k >= 2:
                wdesc(k - 2, h).wait()
            pltpu.sync_copy(acc_sh.at[pl.ds(rbase + k * _BATCH, _BATCH)],
                            wrows[h])
            wdesc(k, h).start()
        wdesc(nwb - 2, 0).wait()
        wdesc(nwb - 1, 1).wait()


def _emb_body(x_ref, w_ref, b_ref, o_ref):
    t = lax.dot_general(x_ref[...].astype(jnp.bfloat16),
                        w_ref[...].astype(jnp.bfloat16),
                        (((1,), (1,)), ((), ())),
                        preferred_element_type=jnp.float32)
    o_ref[...] = t + b_ref[0]


_emb_call = pl.pallas_call(
    _emb_body,
    grid=(_C, _NB),
    in_specs=[
        pl.BlockSpec((_BN, _DIN), lambda c, i: (i, 0)),
        pl.BlockSpec((_CW, _DIN), lambda c, i: (c, 0)),
        pl.BlockSpec((1, 1, _CW), lambda c, i: (c, 0, 0)),
    ],
    out_specs=pl.BlockSpec((_BN, _CW), lambda c, i: (c * _NB + i, 0)),
    out_shape=jax.ShapeDtypeStruct((_C * _N, _CW), jnp.float32),
)


def _gate_norm(agg4, res4, w, g, b, kap):
    t = None
    for c in range(_C):
        p = lax.dot_general(agg4[c].astype(jnp.bfloat16),
                            w[:, c * _CW:(c + 1) * _CW].astype(jnp.bfloat16),
                            (((1,), (1,)), ((), ())),
                            preferred_element_type=jnp.float32)
        t = p if t is None else t + p
    xs = jnp.clip(t, -10.0, 10.0)
    x2 = xs * xs
    gate = jax.nn.sigmoid(kap / (0.5 + 1e-08) - x2 * x2 * 0.01)
    h = t * gate
    res = jnp.concatenate([res4[c] for c in range(_C)], axis=1)
    y = h + res
    mu = jnp.mean(y, axis=1, keepdims=True)
    var = jnp.mean((y - mu) ** 2, axis=1, keepdims=True)
    return (y - mu) / jnp.sqrt(var + 1e-05) * g + b


def _mid_body(agg_ref, res_ref, w_ref, g_ref, b_ref, kap_ref, o_ref):
    hn = _gate_norm(agg_ref[...], res_ref[...], w_ref[...], g_ref[...],
                    b_ref[...], kap_ref[0, 0])
    for c in range(_C):
        o_ref[c] = hn[:, c * _CW:(c + 1) * _CW]


_mid_call = pl.pallas_call(
    _mid_body,
    grid=(_NB,),
    in_specs=[
        pl.BlockSpec((_C, _BN, _CW), lambda i: (0, i, 0)),
        pl.BlockSpec((_C, _BN, _CW), lambda i: (0, i, 0)),
        pl.BlockSpec((_H, _H), lambda i: (0, 0)),
        pl.BlockSpec((1, _H), lambda i: (0, 0)),
        pl.BlockSpec((1, _H), lambda i: (0, 0)),
        pl.BlockSpec((1, 1), lambda i: (0, 0)),
    ],
    out_specs=pl.BlockSpec((_C, _BN, _CW), lambda i: (0, i, 0)),
    out_shape=jax.ShapeDtypeStruct((_C, _N, _CW), jnp.float32),
)


def _final_body(agg_ref, res_ref, w_ref, g_ref, b_ref, kap_ref, ro_ref,
                rob_ref, o_ref):
    hn = _gate_norm(agg_ref[...], res_ref[...], w_ref[...], g_ref[...],
                    b_ref[...], kap_ref[0, 0])
    logits = jnp.sum(hn * ro_ref[...], axis=1, keepdims=True)
    o_ref[...] = jax.nn.sigmoid(logits + rob_ref[0, 0])


_final_call = pl.pallas_call(
    _final_body,
    grid=(_NB,),
    in_specs=[
        pl.BlockSpec((_C, _BN, _CW), lambda i: (0, i, 0)),
        pl.BlockSpec((_C, _BN, _CW), lambda i: (0, i, 0)),
        pl.BlockSpec((_H, _H), lambda i: (0, 0)),
        pl.BlockSpec((1, _H), lambda i: (0, 0)),
        pl.BlockSpec((1, _H), lambda i: (0, 0)),
        pl.BlockSpec((1, 1), lambda i: (0, 0)),
        pl.BlockSpec((1, _H), lambda i: (0, 0)),
        pl.BlockSpec((1, 1), lambda i: (0, 0)),
    ],
    out_specs=pl.BlockSpec((_BN, 1), lambda i: (i, 0)),
    out_shape=jax.ShapeDtypeStruct((_N, 1), jnp.float32),
)


def kernel(x, edge_index, emb_W, emb_b, e8_W, q_W, q_b, k_W, k_b, tau, kappa,
           ln_g, ln_b, ro_W, ro_b):
    src = edge_index[1]
    dst = edge_index[0]
    emb_b4 = emb_b.reshape(_C, 1, _CW)
    g2 = ln_g.reshape(1, _H)
    b2 = ln_b.reshape(1, _H)
    kap2 = kappa.reshape(1, 1)
    rob2 = ro_b.reshape(1, 1)

    h4 = _emb_call(x, emb_W, emb_b4)
    agg = _sc_scatter(h4, src, dst)
    h4b = _mid_call(agg.reshape(_C, _NPAD, _CW), h4.reshape(_C, _N, _CW),
                    e8_W, g2, b2, kap2)
    agg2 = _sc_scatter(h4b.reshape(_C * _N, _CW), src, dst)
    out = _final_call(agg2.reshape(_C, _NPAD, _CW), h4b, e8_W, g2, b2, kap2,
                      ro_W, rob2)
    return out


# speedup vs baseline: 1.0714x; 1.0714x over previous
import functools

import jax
import jax.numpy as jnp
from jax import lax
from jax.experimental import pallas as pl
from jax.experimental.pallas import tpu as pltpu
from jax.experimental.pallas import tpu_sc as plsc

_N = 10000
_E = 160000
_DIN = 256
_H = 512
_C = 4
_CW = _H // _C
_NS = 16
_BATCH = 80
_ZB = 40
_NPAD = 10240
_RPT = _NPAD // _NS
_EPT = _E // _NS
_BN = 2000
_NB = _N // _BN


_sc_mesh = plsc.VectorSubcoreMesh(core_axis_name="c", subcore_axis_name="s")


_NBAT = _EPT // _BATCH


@functools.partial(
    pl.kernel,
    mesh=_sc_mesh,
    out_type=jax.ShapeDtypeStruct((_C * _NPAD, _CW), jnp.float32),
    scratch_types=[
        pltpu.VMEM((_EPT,), jnp.int32),
        pltpu.VMEM((_BATCH,), jnp.int32),
        pltpu.VMEM((_BATCH,), jnp.int32),
        pltpu.VMEM((_BATCH,), jnp.int32),
        pltpu.VMEM((_BATCH, _CW), jnp.float32),
        pltpu.VMEM((_BATCH, _CW), jnp.float32),
        pltpu.VMEM((_BATCH, _CW), jnp.float32),
        pltpu.VMEM_SHARED((_NPAD, _CW), jnp.float32),
        pltpu.SemaphoreType.DMA,
        pltpu.SemaphoreType.DMA,
        pltpu.SemaphoreType.DMA,
        pltpu.SemaphoreType.DMA,
        pltpu.SemaphoreType.DMA,
        pltpu.SemaphoreType.DMA,
        pltpu.SemaphoreType.DMA,
        pltpu.SemaphoreType.DMA,
        pltpu.SemaphoreType.DMA,
        pltpu.SemaphoreType.DMA,
        pltpu.SemaphoreType.DMA,
    ],
)
def _sc_scatter(h4, ei, out, src_off, dst_v0, dst_v1, dst_v2, rows0,
                rows1, rows2, acc_sh, sem0, sem1, sem2, semd0,
                semd1, semd2, sems0, sems1, sems2, semw0, semw1):
    cid = lax.axis_index("c")
    sid = lax.axis_index("s")
    ebase = sid * _EPT
    rbase = sid * _RPT
    z16 = jnp.zeros((16,), jnp.float32)
    slots = ((dst_v0, rows0, sem0, semd0, sems0),
             (dst_v1, rows1, sem1, semd1, sems1),
             (dst_v2, rows2, sem2, semd2, sems2))
    wrows = (rows0, rows1)
    wsems = (semw0, semw1)

    def dcopy(b, dv, sem):
        return pltpu.make_async_copy(
            ei.at[pl.ds(ebase + b * _BATCH, _BATCH)], dv, sem)

    def gcopy(b, rows, sem):
        return pltpu.make_async_copy(
            h4.at[src_off.at[pl.ds(b * _BATCH, _BATCH)]], rows, sem)

    def issue(b, slot):
        dcopy(b, slot[0], slot[3]).start()
        gcopy(b, slot[1], slot[2]).start()

    def drain_scatter(b, slot):
        gcopy(b, slot[1], slot[2]).wait()
        dcopy(b, slot[0], slot[3]).wait()
        pltpu.async_copy(slot[1], acc_sh.at[slot[0]], slot[4], add=True)

    def swait(slot):
        pltpu.make_async_copy(slot[1], acc_sh.at[slot[0]], slot[4]).wait()

    def prep(b, slot):
        swait(slot)
        issue(b, slot)

    for ci in range(2):
        chunk = cid * 2 + ci
        row_off = chunk * _N

        def zrow(i, carry):
            for j in range(_CW // 16):
                rows2[i, pl.ds(j * 16, 16)] = z16
            return carry

        lax.fori_loop(0, _BATCH, zrow, 0)

        def zcopy(k, carry):
            pltpu.sync_copy(rows2,
                            acc_sh.at[pl.ds(rbase + k * _BATCH, _BATCH)])
            return carry

        lax.fori_loop(0, _RPT // _BATCH, zcopy, 0)

        pltpu.sync_copy(ei.at[pl.ds(_E + ebase, _EPT)], src_off)

        def abody(k, carry):
            sl = pl.ds(k * 16, 16)
            src_off[sl] = src_off[sl] + row_off
            return carry

        lax.fori_loop(0, _EPT // 16, abody, 0)
        plsc.subcore_barrier()

        issue(0, slots[0])
        issue(1, slots[1])
        issue(2, slots[2])

        def pbody(p, carry):
            b0 = 3 * p
            drain_scatter(b0, slots[0])
            prep(b0 + 3, slots[0])
            drain_scatter(b0 + 1, slots[1])
            prep(b0 + 4, slots[1])
            drain_scatter(b0 + 2, slots[2])
            prep(b0 + 5, slots[2])
            return carry

        lax.fori_loop(0, (_NBAT - 5) // 3, pbody, 0)
        drain_scatter(_NBAT - 5, slots[0])
        prep(_NBAT - 2, slots[0])
        drain_scatter(_NBAT - 4, slots[1])
        prep(_NBAT - 1, slots[1])
        drain_scatter(_NBAT - 3, slots[2])
        drain_scatter(_NBAT - 2, slots[0])
        drain_scatter(_NBAT - 1, slots[1])
        for sl in slots:
            swait(sl)
        plsc.subcore_barrier()

        woff = chunk * _NPAD + rbase
        nwb = _RPT // _BATCH

        def wdesc(k, h):
            return pltpu.make_async_copy(
                wrows[h], out.at[pl.ds(woff + k * _BATCH, _BATCH)], wsems[h])

        for k in range(nwb):
            h = k % 2
            if k >= 2:
                wdesc(k - 2, h).wait()
            pltpu.sync_copy(acc_sh.at[pl.ds(rbase + k * _BATCH, _BATCH)],
                            wrows[h])
            wdesc(k, h).start()
        wdesc(nwb - 2, 0).wait()
        wdesc(nwb - 1, 1).wait()


def _emb_body(x_ref, w_ref, b_ref, o_ref):
    t = lax.dot_general(x_ref[...].astype(jnp.bfloat16),
                        w_ref[...].astype(jnp.bfloat16),
                        (((1,), (1,)), ((), ())),
                        preferred_element_type=jnp.float32)
    t = t + b_ref[...]
    for c in range(_C):
        o_ref[c] = t[:, c * _CW:(c + 1) * _CW]


_emb_call = pl.pallas_call(
    _emb_body,
    grid=(_NB,),
    in_specs=[
        pl.BlockSpec((_BN, _DIN), lambda i: (i, 0)),
        pl.BlockSpec((_H, _DIN), lambda i: (0, 0)),
        pl.BlockSpec((1, _H), lambda i: (0, 0)),
    ],
    out_specs=pl.BlockSpec((_C, _BN, _CW), lambda i: (0, i, 0)),
    out_shape=jax.ShapeDtypeStruct((_C, _N, _CW), jnp.float32),
)


def _gate_norm(agg4, res4, w, g, b, kap):
    t = None
    for c in range(_C):
        p = lax.dot_general(agg4[c].astype(jnp.bfloat16),
                            w[:, c * _CW:(c + 1) * _CW].astype(jnp.bfloat16),
                            (((1,), (1,)), ((), ())),
                            preferred_element_type=jnp.float32)
        t = p if t is None else t + p
    xs = jnp.clip(t, -10.0, 10.0)
    x2 = xs * xs
    gate = jax.nn.sigmoid(kap / (0.5 + 1e-08) - x2 * x2 * 0.01)
    h = t * gate
    res = jnp.concatenate([res4[c] for c in range(_C)], axis=1)
    y = h + res
    mu = jnp.mean(y, axis=1, keepdims=True)
    var = jnp.mean((y - mu) ** 2, axis=1, keepdims=True)
    return (y - mu) / jnp.sqrt(var + 1e-05) * g + b


def _mid_body(agg_ref, res_ref, w_ref, g_ref, b_ref, kap_ref, o_ref):
    hn = _gate_norm(agg_ref[...], res_ref[...], w_ref[...], g_ref[...],
                    b_ref[...], kap_ref[0, 0])
    for c in range(_C):
        o_ref[c] = hn[:, c * _CW:(c + 1) * _CW]


_mid_call = pl.pallas_call(
    _mid_body,
    grid=(_NB,),
    in_specs=[
        pl.BlockSpec((_C, _BN, _CW), lambda i: (0, i, 0)),
        pl.BlockSpec((_C, _BN, _CW), lambda i: (0, i, 0)),
        pl.BlockSpec((_H, _H), lambda i: (0, 0)),
        pl.BlockSpec((1, _H), lambda i: (0, 0)),
        pl.BlockSpec((1, _H), lambda i: (0, 0)),
        pl.BlockSpec((1, 1), lambda i: (0, 0)),
    ],
    out_specs=pl.BlockSpec((_C, _BN, _CW), lambda i: (0, i, 0)),
    out_shape=jax.ShapeDtypeStruct((_C, _N, _CW), jnp.float32),
)


def _final_body(agg_ref, res_ref, w_ref, g_ref, b_ref, kap_ref, ro_ref,
                rob_ref, o_ref):
    hn = _gate_norm(agg_ref[...], res_ref[...], w_ref[...], g_ref[...],
                    b_ref[...], kap_ref[0, 0])
    logits = jnp.sum(hn * ro_ref[...], axis=1, keepdims=True)
    o_ref[...] = jax.nn.sigmoid(logits + rob_ref[0, 0])


_final_call = pl.pallas_call(
    _final_body,
    grid=(_NB,),
    in_specs=[
        pl.BlockSpec((_C, _BN, _CW), lambda i: (0, i, 0)),
        pl.BlockSpec((_C, _BN, _CW), lambda i: (0, i, 0)),
        pl.BlockSpec((_H, _H), lambda i: (0, 0)),
        pl.BlockSpec((1, _H), lambda i: (0, 0)),
        pl.BlockSpec((1, _H), lambda i: (0, 0)),
        pl.BlockSpec((1, 1), lambda i: (0, 0)),
        pl.BlockSpec((1, _H), lambda i: (0, 0)),
        pl.BlockSpec((1, 1), lambda i: (0, 0)),
    ],
    out_specs=pl.BlockSpec((_BN, 1), lambda i: (i, 0)),
    out_shape=jax.ShapeDtypeStruct((_N, 1), jnp.float32),
)


def kernel(x, edge_index, emb_W, emb_b, e8_W, q_W, q_b, k_W, k_b, tau, kappa,
           ln_g, ln_b, ro_W, ro_b):
    ei = edge_index.reshape(-1)
    emb_b2 = emb_b.reshape(1, _H)
    g2 = ln_g.reshape(1, _H)
    b2 = ln_b.reshape(1, _H)
    kap2 = kappa.reshape(1, 1)
    rob2 = ro_b.reshape(1, 1)

    h4 = _emb_call(x, emb_W, emb_b2)
    agg = _sc_scatter(h4.reshape(_C * _N, _CW), ei)
    h4b = _mid_call(agg.reshape(_C, _NPAD, _CW), h4,
                    e8_W, g2, b2, kap2)
    agg2 = _sc_scatter(h4b.reshape(_C * _N, _CW), ei)
    out = _final_call(agg2.reshape(_C, _NPAD, _CW), h4b, e8_W, g2, b2, kap2,
                      ro_W, rob2)
    return out


# speedup vs baseline: 1.0715x; 1.0001x over previous
import functools

import jax
import jax.numpy as jnp
from jax import lax
from jax.experimental import pallas as pl
from jax.experimental.pallas import tpu as pltpu
from jax.experimental.pallas import tpu_sc as plsc

_N = 10000
_E = 160000
_DIN = 256
_H = 512
_C = 4
_CW = _H // _C
_NS = 16
_BATCH = 80
_NPAD = 10240
_RPT = _NPAD // _NS
_EPT = _E // _NS
_BN = 2000
_NB = _N // _BN


_sc_mesh = plsc.VectorSubcoreMesh(core_axis_name="c", subcore_axis_name="s")


_NBAT = _EPT // _BATCH


@functools.partial(
    pl.kernel,
    mesh=_sc_mesh,
    out_type=jax.ShapeDtypeStruct((_C * _NPAD, _CW), jnp.float32),
    scratch_types=[
        pltpu.VMEM((_EPT,), jnp.int32),
        pltpu.VMEM((_BATCH,), jnp.int32),
        pltpu.VMEM((_BATCH,), jnp.int32),
        pltpu.VMEM((_BATCH,), jnp.int32),
        pltpu.VMEM((_BATCH, _CW), jnp.float32),
        pltpu.VMEM((_BATCH, _CW), jnp.float32),
        pltpu.VMEM((_BATCH, _CW), jnp.float32),
        pltpu.VMEM_SHARED((_NPAD, _CW), jnp.float32),
        pltpu.SemaphoreType.DMA,
        pltpu.SemaphoreType.DMA,
        pltpu.SemaphoreType.DMA,
        pltpu.SemaphoreType.DMA,
        pltpu.SemaphoreType.DMA,
        pltpu.SemaphoreType.DMA,
        pltpu.SemaphoreType.DMA,
        pltpu.SemaphoreType.DMA,
        pltpu.SemaphoreType.DMA,
        pltpu.SemaphoreType.DMA,
        pltpu.SemaphoreType.DMA,
    ],
)
def _sc_scatter(h4, ei, out, src_off, dst_v0, dst_v1, dst_v2, rows0,
                rows1, rows2, acc_sh, sem0, sem1, sem2, semd0,
                semd1, semd2, sems0, sems1, sems2, semw0, semw1):
    cid = lax.axis_index("c")
    sid = lax.axis_index("s")
    ebase = sid * _EPT
    rbase = sid * _RPT
    z16 = jnp.zeros((16,), jnp.float32)
    slots = ((dst_v0, rows0, sem0, semd0, sems0),
             (dst_v1, rows1, sem1, semd1, sems1),
             (dst_v2, rows2, sem2, semd2, sems2))
    wrows = (rows0, rows1)
    wsems = (semw0, semw1)

    def dcopy(b, dv, sem):
        return pltpu.make_async_copy(
            ei.at[pl.ds(ebase + b * _BATCH, _BATCH)], dv, sem)

    def gcopy(b, rows, sem):
        return pltpu.make_async_copy(
            h4.at[src_off.at[pl.ds(b * _BATCH, _BATCH)]], rows, sem)

    def issue(b, slot):
        dcopy(b, slot[0], slot[3]).start()
        gcopy(b, slot[1], slot[2]).start()

    def drain_scatter(b, slot):
        gcopy(b, slot[1], slot[2]).wait()
        dcopy(b, slot[0], slot[3]).wait()
        pltpu.async_copy(slot[1], acc_sh.at[slot[0]], slot[4], add=True)

    def swait(slot):
        pltpu.make_async_copy(slot[1], acc_sh.at[slot[0]], slot[4]).wait()

    def prep(b, slot):
        swait(slot)
        issue(b, slot)

    for ci in range(2):
        chunk = cid * 2 + ci
        row_off = chunk * _N

        def zrow(i, carry):
            for j in range(_CW // 16):
                rows2[i, pl.ds(j * 16, 16)] = z16
            return carry

        lax.fori_loop(0, _BATCH, zrow, 0)

        def zcopy(k, carry):
            pltpu.sync_copy(rows2,
                            acc_sh.at[pl.ds(rbase + k * _BATCH, _BATCH)])
            return carry

        lax.fori_loop(0, _RPT // _BATCH, zcopy, 0)

        pltpu.sync_copy(ei.at[pl.ds(_E + ebase, _EPT)], src_off)

        def abody(k, carry):
            sl = pl.ds(k * 16, 16)
            src_off[sl] = src_off[sl] + row_off
            return carry

        lax.fori_loop(0, _EPT // 16, abody, 0)
        plsc.subcore_barrier()

        issue(0, slots[0])
        issue(1, slots[1])
        issue(2, slots[2])

        def pbody(p, carry):
            b0 = 3 * p
            drain_scatter(b0, slots[0])
            prep(b0 + 3, slots[0])
            drain_scatter(b0 + 1, slots[1])
            prep(b0 + 4, slots[1])
            drain_scatter(b0 + 2, slots[2])
            prep(b0 + 5, slots[2])
            return carry

        lax.fori_loop(0, (_NBAT - 5) // 3, pbody, 0)
        drain_scatter(_NBAT - 5, slots[0])
        prep(_NBAT - 2, slots[0])
        drain_scatter(_NBAT - 4, slots[1])
        prep(_NBAT - 1, slots[1])
        drain_scatter(_NBAT - 3, slots[2])
        drain_scatter(_NBAT - 2, slots[0])
        drain_scatter(_NBAT - 1, slots[1])
        for sl in slots:
            swait(sl)
        plsc.subcore_barrier()

        woff = chunk * _NPAD + rbase
        nwb = _RPT // _BATCH

        def wdesc(k, h):
            return pltpu.make_async_copy(
                wrows[h], out.at[pl.ds(woff + k * _BATCH, _BATCH)], wsems[h])

        for k in range(nwb):
            h = k % 2
            if k >= 2:
                wdesc(k - 2, h).wait()
            pltpu.sync_copy(acc_sh.at[pl.ds(rbase + k * _BATCH, _BATCH)],
                            wrows[h])
            wdesc(k, h).start()
        wdesc(nwb - 2, 0).wait()
        wdesc(nwb - 1, 1).wait()


def _emb_body(x_ref, w_ref, b_ref, o_ref):
    t = lax.dot_general(x_ref[...].astype(jnp.bfloat16),
                        w_ref[...].astype(jnp.bfloat16),
                        (((1,), (1,)), ((), ())),
                        preferred_element_type=jnp.float32)
    t = t + b_ref[...]
    for c in range(_C):
        o_ref[c] = t[:, c * _CW:(c + 1) * _CW]


_emb_call = pl.pallas_call(
    _emb_body,
    grid=(_NB,),
    in_specs=[
        pl.BlockSpec((_BN, _DIN), lambda i: (i, 0)),
        pl.BlockSpec((_H, _DIN), lambda i: (0, 0)),
        pl.BlockSpec((1, _H), lambda i: (0, 0)),
    ],
    out_specs=pl.BlockSpec((_C, _BN, _CW), lambda i: (0, i, 0)),
    out_shape=jax.ShapeDtypeStruct((_C, _N, _CW), jnp.float32),
)


def _gate_norm(agg4, res4, w, g, b, kap):
    t = None
    for c in range(_C):
        p = lax.dot_general(agg4[c].astype(jnp.bfloat16),
                            w[:, c * _CW:(c + 1) * _CW].astype(jnp.bfloat16),
                            (((1,), (1,)), ((), ())),
                            preferred_element_type=jnp.float32)
        t = p if t is None else t + p
    xs = jnp.clip(t, -10.0, 10.0)
    x2 = xs * xs
    gate = jax.nn.sigmoid(kap / (0.5 + 1e-08) - x2 * x2 * 0.01)
    h = t * gate
    res = jnp.concatenate([res4[c] for c in range(_C)], axis=1)
    y = h + res
    mu = jnp.mean(y, axis=1, keepdims=True)
    var = jnp.mean((y - mu) ** 2, axis=1, keepdims=True)
    return (y - mu) / jnp.sqrt(var + 1e-05) * g + b


def _mid_body(agg_ref, res_ref, w_ref, g_ref, b_ref, kap_ref, o_ref):
    hn = _gate_norm(agg_ref[...], res_ref[...], w_ref[...], g_ref[...],
                    b_ref[...], kap_ref[0, 0])
    for c in range(_C):
        o_ref[c] = hn[:, c * _CW:(c + 1) * _CW]


_mid_call = pl.pallas_call(
    _mid_body,
    grid=(_NB,),
    in_specs=[
        pl.BlockSpec((_C, _BN, _CW), lambda i: (0, i, 0)),
        pl.BlockSpec((_C, _BN, _CW), lambda i: (0, i, 0)),
        pl.BlockSpec((_H, _H), lambda i: (0, 0)),
        pl.BlockSpec((1, _H), lambda i: (0, 0)),
        pl.BlockSpec((1, _H), lambda i: (0, 0)),
        pl.BlockSpec((1, 1), lambda i: (0, 0)),
    ],
    out_specs=pl.BlockSpec((_C, _BN, _CW), lambda i: (0, i, 0)),
    out_shape=jax.ShapeDtypeStruct((_C, _N, _CW), jnp.float32),
)


def _final_body(agg_ref, res_ref, w_ref, g_ref, b_ref, kap_ref, ro_ref,
                rob_ref, o_ref):
    hn = _gate_norm(agg_ref[...], res_ref[...], w_ref[...], g_ref[...],
                    b_ref[...], kap_ref[0, 0])
    logits = jnp.sum(hn * ro_ref[...], axis=1, keepdims=True)
    o_ref[...] = jax.nn.sigmoid(logits + rob_ref[0, 0])


_final_call = pl.pallas_call(
    _final_body,
    grid=(_NB,),
    in_specs=[
        pl.BlockSpec((_C, _BN, _CW), lambda i: (0, i, 0)),
        pl.BlockSpec((_C, _BN, _CW), lambda i: (0, i, 0)),
        pl.BlockSpec((_H, _H), lambda i: (0, 0)),
        pl.BlockSpec((1, _H), lambda i: (0, 0)),
        pl.BlockSpec((1, _H), lambda i: (0, 0)),
        pl.BlockSpec((1, 1), lambda i: (0, 0)),
        pl.BlockSpec((1, _H), lambda i: (0, 0)),
        pl.BlockSpec((1, 1), lambda i: (0, 0)),
    ],
    out_specs=pl.BlockSpec((_BN, 1), lambda i: (i, 0)),
    out_shape=jax.ShapeDtypeStruct((_N, 1), jnp.float32),
)


def kernel(x, edge_index, emb_W, emb_b, e8_W, q_W, q_b, k_W, k_b, tau, kappa,
           ln_g, ln_b, ro_W, ro_b):
    ei = edge_index.reshape(-1)
    emb_b2 = emb_b.reshape(1, _H)
    g2 = ln_g.reshape(1, _H)
    b2 = ln_b.reshape(1, _H)
    kap2 = kappa.reshape(1, 1)
    rob2 = ro_b.reshape(1, 1)

    h4 = _emb_call(x, emb_W, emb_b2)
    agg = _sc_scatter(h4.reshape(_C * _N, _CW), ei)
    h4b = _mid_call(agg.reshape(_C, _NPAD, _CW), h4,
                    e8_W, g2, b2, kap2)
    agg2 = _sc_scatter(h4b.reshape(_C * _N, _CW), ei)
    out = _final_call(agg2.reshape(_C, _NPAD, _CW), h4b, e8_W, g2, b2, kap2,
                      ro_W, rob2)
    return out
